# baseline (speedup 1.0000x reference)
import functools

import jax
import jax.numpy as jnp
from jax import lax
from jax.experimental import pallas as pl
from jax.experimental.pallas import tpu as pltpu
from jax.experimental.pallas import tpu_sc as plsc

N_NODES = 10000
N_EDGES = 320000
D = 128
H = 128

EB = 128
NB = N_EDGES // EB


def _edge_body(ef_ref, gs_ref, gd_ref, w1_ref, b1_ref, w2_ref, b2_ref,
               gm_ref, bt_ref, out_ref):
    x = ef_ref[...]
    h = jnp.dot(x.astype(jnp.bfloat16), w1_ref[...].astype(jnp.bfloat16),
                preferred_element_type=jnp.float32)
    h = h + gs_ref[...].astype(jnp.float32) + gd_ref[...].astype(jnp.float32)
    h = h + b1_ref[...]
    h = h * jax.nn.sigmoid(h)
    y = jnp.dot(h.astype(jnp.bfloat16), w2_ref[...].astype(jnp.bfloat16),
                preferred_element_type=jnp.float32) + b2_ref[...]
    mu = jnp.mean(y, axis=-1, keepdims=True)
    var = jnp.mean((y - mu) ** 2, axis=-1, keepdims=True)
    y = (y - mu) * lax.rsqrt(var + 1e-5)
    out_ref[...] = y * gm_ref[...] + bt_ref[...] + x


def _edge_mlp(ef, gs, gd, w1e, b1, w2, b2, gamma, beta, blk=2000):
    grid = (N_EDGES // blk,)
    row = lambda i: (i, 0)
    fix = lambda i: (0, 0)
    return pl.pallas_call(
        _edge_body,
        grid=grid,
        in_specs=[
            pl.BlockSpec((blk, D), row),
            pl.BlockSpec((blk, H), row),
            pl.BlockSpec((blk, H), row),
            pl.BlockSpec((D, H), fix),
            pl.BlockSpec((1, H), fix),
            pl.BlockSpec((H, D), fix),
            pl.BlockSpec((1, D), fix),
            pl.BlockSpec((1, D), fix),
            pl.BlockSpec((1, D), fix),
        ],
        out_specs=pl.BlockSpec((blk, D), row),
        out_shape=jax.ShapeDtypeStruct((N_EDGES, D), jnp.float32),
        compiler_params=pltpu.CompilerParams(
            dimension_semantics=("arbitrary",)),
    )(ef, gs, gd, w1e, b1, w2, b2, gamma, beta)


def _node_body(nf_ref, agg_ref, w1n_ref, w1a_ref, b1_ref, w2_ref, b2_ref,
               gm_ref, bt_ref, w1s_ref, w1d_ref,
               nf_out, ps_out, pd_out):
    x = nf_ref[...]
    a = agg_ref[0] + agg_ref[1]
    h = (jnp.dot(x, w1n_ref[...], preferred_element_type=jnp.float32)
         + jnp.dot(a, w1a_ref[...], preferred_element_type=jnp.float32)
         + b1_ref[...])
    h = h * jax.nn.sigmoid(h)
    y = jnp.dot(h, w2_ref[...], preferred_element_type=jnp.float32) + b2_ref[...]
    mu = jnp.mean(y, axis=-1, keepdims=True)
    var = jnp.mean((y - mu) ** 2, axis=-1, keepdims=True)
    y = (y - mu) * lax.rsqrt(var + 1e-5)
    y = y * gm_ref[...] + bt_ref[...] + x
    nf_out[...] = y
    ps_out[...] = jnp.dot(y, w1s_ref[...], preferred_element_type=jnp.float32)
    pd_out[...] = jnp.dot(y, w1d_ref[...], preferred_element_type=jnp.float32)


def _node_mlp(nf, agg2, w1n, w1a, b1, w2, b2, gamma, beta, w1s_nxt, w1d_nxt):
    out_shape = [
        jax.ShapeDtypeStruct((N_NODES, D), jnp.float32),
        jax.ShapeDtypeStruct((N_NODES, H), jnp.float32),
        jax.ShapeDtypeStruct((N_NODES, H), jnp.float32),
    ]
    return pl.pallas_call(_node_body, out_shape=out_shape)(
        nf, agg2, w1n, w1a, b1, w2, b2, gamma, beta, w1s_nxt, w1d_nxt)


def _proj_body(nf_ref, w1s_ref, w1d_ref, ps_out, pd_out):
    x = nf_ref[...]
    ps_out[...] = jnp.dot(x, w1s_ref[...], preferred_element_type=jnp.float32)
    pd_out[...] = jnp.dot(x, w1d_ref[...], preferred_element_type=jnp.float32)


def _proj(nf, w1s, w1d):
    out_shape = [
        jax.ShapeDtypeStruct((N_NODES, H), jnp.float32),
        jax.ShapeDtypeStruct((N_NODES, H), jnp.float32),
    ]
    return pl.pallas_call(_proj_body, out_shape=out_shape)(nf, w1s, w1d)


def _make_gather():
    info = plsc.get_sparse_core_info()
    nc, ns = info.num_cores, info.num_subcores
    nw = nc * ns
    mesh = plsc.VectorSubcoreMesh(core_axis_name="c", subcore_axis_name="s")

    @functools.partial(
        pl.kernel,
        mesh=mesh,
        out_type=(
            jax.ShapeDtypeStruct((N_EDGES, H), jnp.float32),
            jax.ShapeDtypeStruct((N_EDGES, H), jnp.float32),
        ),
        scratch_types=[
            pltpu.VMEM((EB,), jnp.int32),
            pltpu.VMEM((EB,), jnp.int32),
            pltpu.VMEM((EB, H), jnp.float32),
            pltpu.VMEM((EB, H), jnp.float32),
            pltpu.SemaphoreType.DMA,
            pltpu.SemaphoreType.DMA,
        ],
    )
    def gather(ps_hbm, pd_hbm, src_hbm, dst_hbm, gs_hbm, gd_hbm,
               si, di, bs, bd, sem_s, sem_d):
        wid = lax.axis_index("s") * nc + lax.axis_index("c")
        n_my = NB // nw + jnp.where(wid < NB % nw, 1, 0)

        def body(j, _):
            base = (j * nw + wid) * EB
            pltpu.sync_copy(src_hbm.at[pl.ds(base, EB)], si)
            pltpu.sync_copy(dst_hbm.at[pl.ds(base, EB)], di)
            cs = pltpu.async_copy(ps_hbm.at[si], bs, sem_s)
            cd = pltpu.async_copy(pd_hbm.at[di], bd, sem_d)
            cs.wait()
            cd.wait()
            pltpu.sync_copy(bs, gs_hbm.at[pl.ds(base, EB)])
            pltpu.sync_copy(bd, gd_hbm.at[pl.ds(base, EB)])
            return 0

        lax.fori_loop(0, n_my, body, 0)

    return gather


def _make_scatter():
    info = plsc.get_sparse_core_info()
    nc, ns = info.num_cores, info.num_subcores
    nw = nc * ns
    rps = 624
    tail = N_NODES - rps * ns
    mesh = plsc.VectorSubcoreMesh(core_axis_name="c", subcore_axis_name="s")

    @functools.partial(
        pl.kernel,
        mesh=mesh,
        out_type=jax.ShapeDtypeStruct((2, N_NODES, D), jnp.float32),
        scratch_types=[
            pltpu.VMEM((EB,), jnp.int32),
            pltpu.VMEM((EB, D), jnp.float32),
            pltpu.VMEM_SHARED((N_NODES, D), jnp.float32),
        ],
    )
    def scatter(e_hbm, dst_hbm, zeros_hbm, out_hbm, di, rows, acc):
        cid = lax.axis_index("c")
        sid = lax.axis_index("s")
        wid = sid * nc + cid
        r0 = sid * rps
        pltpu.sync_copy(zeros_hbm.at[pl.ds(r0, rps)], acc.at[pl.ds(r0, rps)])

        @pl.when(sid == 0)
        def _():
            pltpu.sync_copy(zeros_hbm.at[pl.ds(rps * ns, tail)],
                            acc.at[pl.ds(rps * ns, tail)])

        plsc.subcore_barrier()

        n_my = NB // nw + jnp.where(wid < NB % nw, 1, 0)

        def body(j, _):
            base = (j * nw + wid) * EB
            pltpu.sync_copy(dst_hbm.at[pl.ds(base, EB)], di)
            pltpu.sync_copy(e_hbm.at[pl.ds(base, EB)], rows)
            pltpu.sync_copy(rows, acc.at[di], add=True)
            return 0

        lax.fori_loop(0, n_my, body, 0)
        plsc.subcore_barrier()
        pltpu.sync_copy(acc.at[pl.ds(r0, rps)],
                        out_hbm.at[cid, pl.ds(r0, rps)])

        @pl.when(sid == 0)
        def _():
            pltpu.sync_copy(acc.at[pl.ds(rps * ns, tail)],
                            out_hbm.at[cid, pl.ds(rps * ns, tail)])

    return scatter


def kernel(efeat, nfeat, edge_index, params):
    src = edge_index[0].astype(jnp.int32)
    dst = edge_index[1].astype(jnp.int32)

    gather = _make_gather()
    scatter = _make_scatter()
    zeros = jnp.zeros((N_NODES, D), jnp.float32)

    def prep(p):
        e, n = p['edge'], p['node']
        return dict(
            w1e=e['w1'][:D], w1s=e['w1'][D:2 * D], w1d=e['w1'][2 * D:],
            eb1=e['b1'].reshape(1, H), ew2=e['w2'],
            eb2=e['b2'].reshape(1, D), eg=e['gamma'].reshape(1, D),
            ebt=e['beta'].reshape(1, D),
            w1n=n['w1'][:D], w1a=n['w1'][D:],
            nb1=n['b1'].reshape(1, H), nw2=n['w2'],
            nb2=n['b2'].reshape(1, D), ng=n['gamma'].reshape(1, D),
            nbt=n['beta'].reshape(1, D),
        )

    ps_list = [prep(p) for p in params]
    nlayers = len(ps_list)

    ps, pd = _proj(nfeat, ps_list[0]['w1s'], ps_list[0]['w1d'])
    for l, q in enumerate(ps_list):
        gs, gd = gather(ps, pd, src, dst)
        efeat = _edge_mlp(efeat, gs, gd, q['w1e'], q['eb1'], q['ew2'],
                          q['eb2'], q['eg'], q['ebt'])
        agg2 = scatter(efeat, dst, zeros)
        nxt = ps_list[(l + 1) % nlayers]
        nfeat, ps, pd = _node_mlp(nfeat, agg2, q['w1n'], q['w1a'], q['nb1'],
                                  q['nw2'], q['nb2'], q['ng'], q['nbt'],
                                  nxt['w1s'], nxt['w1d'])
    return (efeat, nfeat)


# speedup vs baseline: 1.3740x; 1.3740x over previous
import functools

import jax
import jax.numpy as jnp
from jax import lax
from jax.experimental import pallas as pl
from jax.experimental.pallas import tpu as pltpu
from jax.experimental.pallas import tpu_sc as plsc

N_NODES = 10000
N_EDGES = 320000
D = 128
H = 128

EB = 128
NB = N_EDGES // EB


def _edge_body(ef_ref, gs_ref, gd_ref, w1_ref, b1_ref, w2_ref, b2_ref,
               gm_ref, bt_ref, out_ref):
    x = ef_ref[...]
    h = jnp.dot(x, w1_ref[...], preferred_element_type=jnp.float32)
    h = h + gs_ref[...] + gd_ref[...] + b1_ref[...]
    h = h * jax.nn.sigmoid(h)
    y = jnp.dot(h, w2_ref[...], preferred_element_type=jnp.float32) + b2_ref[...]
    mu = jnp.mean(y, axis=-1, keepdims=True)
    var = jnp.mean((y - mu) ** 2, axis=-1, keepdims=True)
    y = (y - mu) * lax.rsqrt(var + 1e-5)
    out_ref[...] = y * gm_ref[...] + bt_ref[...] + x


def _edge_mlp(ef, gs, gd, w1e, b1, w2, b2, gamma, beta, blk=2000):
    grid = (N_EDGES // blk,)
    row = lambda i: (i, 0)
    fix = lambda i: (0, 0)
    return pl.pallas_call(
        _edge_body,
        grid=grid,
        in_specs=[
            pl.BlockSpec((blk, D), row),
            pl.BlockSpec((blk, H), row),
            pl.BlockSpec((blk, H), row),
            pl.BlockSpec((D, H), fix),
            pl.BlockSpec((1, H), fix),
            pl.BlockSpec((H, D), fix),
            pl.BlockSpec((1, D), fix),
            pl.BlockSpec((1, D), fix),
            pl.BlockSpec((1, D), fix),
        ],
        out_specs=pl.BlockSpec((blk, D), row),
        out_shape=jax.ShapeDtypeStruct((N_EDGES, D), jnp.float32),
        compiler_params=pltpu.CompilerParams(
            dimension_semantics=("arbitrary",)),
    )(ef, gs, gd, w1e, b1, w2, b2, gamma, beta)


def _node_body(nf_ref, agg_ref, w1n_ref, w1a_ref, b1_ref, w2_ref, b2_ref,
               gm_ref, bt_ref, w1s_ref, w1d_ref,
               nf_out, ps_out, pd_out):
    x = nf_ref[...]
    a = agg_ref[0] + agg_ref[1]
    h = (jnp.dot(x, w1n_ref[...], preferred_element_type=jnp.float32)
         + jnp.dot(a, w1a_ref[...], preferred_element_type=jnp.float32)
         + b1_ref[...])
    h = h * jax.nn.sigmoid(h)
    y = jnp.dot(h, w2_ref[...], preferred_element_type=jnp.float32) + b2_ref[...]
    mu = jnp.mean(y, axis=-1, keepdims=True)
    var = jnp.mean((y - mu) ** 2, axis=-1, keepdims=True)
    y = (y - mu) * lax.rsqrt(var + 1e-5)
    y = y * gm_ref[...] + bt_ref[...] + x
    nf_out[...] = y
    ps_out[...] = jnp.dot(y, w1s_ref[...], preferred_element_type=jnp.float32)
    pd_out[...] = jnp.dot(y, w1d_ref[...], preferred_element_type=jnp.float32)


def _node_mlp(nf, agg2, w1n, w1a, b1, w2, b2, gamma, beta, w1s_nxt, w1d_nxt):
    out_shape = [
        jax.ShapeDtypeStruct((N_NODES, D), jnp.float32),
        jax.ShapeDtypeStruct((N_NODES, H), jnp.float32),
        jax.ShapeDtypeStruct((N_NODES, H), jnp.float32),
    ]
    return pl.pallas_call(_node_body, out_shape=out_shape)(
        nf, agg2, w1n, w1a, b1, w2, b2, gamma, beta, w1s_nxt, w1d_nxt)


def _proj_body(nf_ref, w1s_ref, w1d_ref, ps_out, pd_out):
    x = nf_ref[...]
    ps_out[...] = jnp.dot(x, w1s_ref[...], preferred_element_type=jnp.float32)
    pd_out[...] = jnp.dot(x, w1d_ref[...], preferred_element_type=jnp.float32)


def _proj(nf, w1s, w1d):
    out_shape = [
        jax.ShapeDtypeStruct((N_NODES, H), jnp.float32),
        jax.ShapeDtypeStruct((N_NODES, H), jnp.float32),
    ]
    return pl.pallas_call(_proj_body, out_shape=out_shape)(nf, w1s, w1d)


def _make_gather():
    info = plsc.get_sparse_core_info()
    ns = info.num_subcores
    mesh = plsc.VectorSubcoreMesh(core_axis_name="c", subcore_axis_name="s")
    GB = 400
    E_PER = N_EDGES // ns
    NIT = E_PER // GB
    SUB = ((0, 128), (128, 128), (256, 128), (384, 16))

    @functools.partial(
        pl.kernel,
        mesh=mesh,
        out_type=(
            jax.ShapeDtypeStruct((N_EDGES, H), jnp.float32),
            jax.ShapeDtypeStruct((N_EDGES, H), jnp.float32),
        ),
        scratch_types=[
            pltpu.VMEM((E_PER,), jnp.int32),
            pltpu.VMEM((GB, H), jnp.float32),
            pltpu.VMEM((GB, H), jnp.float32),
            pltpu.SemaphoreType.DMA,
            pltpu.SemaphoreType.DMA,
            pltpu.SemaphoreType.DMA,
            pltpu.SemaphoreType.DMA,
        ],
    )
    def gather(ps_hbm, pd_hbm, src_hbm, dst_hbm, gs_hbm, gd_hbm,
               ibig, buf0, buf1, g0, g1, w0, w1):
        cid = lax.axis_index("c")
        sid = lax.axis_index("s")
        base_e = sid * E_PER

        def run(idx_hbm, tab_hbm, out_hbm):
            pltpu.sync_copy(idx_hbm.at[pl.ds(base_e, E_PER)], ibig)

            def fire(it, buf, gsem):
                off = it * GB
                for (o, n) in SUB:
                    pltpu.async_copy(tab_hbm.at[ibig.at[pl.ds(off + o, n)]],
                                     buf.at[pl.ds(o, n)], gsem)

            def wait_g(buf, gsem):
                for (o, n) in SUB:
                    pltpu.make_async_copy(tab_hbm.at[pl.ds(0, n)],
                                          buf.at[pl.ds(o, n)], gsem).wait()

            def write(it, buf, wsem):
                pltpu.async_copy(
                    buf, out_hbm.at[pl.ds(base_e + it * GB, GB)], wsem)

            def wait_w(buf, wsem):
                pltpu.make_async_copy(out_hbm.at[pl.ds(base_e, GB)],
                                      buf, wsem).wait()

            fire(0, buf0, g0)

            def body(i, _):
                @pl.when(i > 0)
                def _():
                    wait_w(buf1, w1)

                fire(2 * i + 1, buf1, g1)
                wait_g(buf0, g0)
                write(2 * i, buf0, w0)

                @pl.when(i < NIT // 2 - 1)
                def _():
                    wait_w(buf0, w0)
                    fire(2 * i + 2, buf0, g0)

                wait_g(buf1, g1)
                write(2 * i + 1, buf1, w1)
                return 0

            lax.fori_loop(0, NIT // 2, body, 0)
            wait_w(buf0, w0)
            wait_w(buf1, w1)

        @pl.when(cid == 0)
        def _():
            run(src_hbm, ps_hbm, gs_hbm)

        @pl.when(cid == 1)
        def _():
            run(dst_hbm, pd_hbm, gd_hbm)

    return gather


def _make_scatter():
    info = plsc.get_sparse_core_info()
    nc, ns = info.num_cores, info.num_subcores
    nw = nc * ns
    rps = 624
    tail = N_NODES - rps * ns
    mesh = plsc.VectorSubcoreMesh(core_axis_name="c", subcore_axis_name="s")

    blk_per_w = NB // nw
    n_extra = NB - blk_per_w * nw

    @functools.partial(
        pl.kernel,
        mesh=mesh,
        out_type=jax.ShapeDtypeStruct((2, N_NODES, D), jnp.float32),
        scratch_types=[
            pltpu.VMEM((EB,), jnp.int32),
            pltpu.VMEM((EB,), jnp.int32),
            pltpu.VMEM((EB, D), jnp.float32),
            pltpu.VMEM((EB, D), jnp.float32),
            pltpu.VMEM_SHARED((N_NODES, D), jnp.float32),
            pltpu.SemaphoreType.DMA,
            pltpu.SemaphoreType.DMA,
        ],
    )
    def scatter(e_hbm, dst_hbm, zeros_hbm, out_hbm,
                di0, di1, rb0, rb1, acc, r0sem, r1sem):
        cid = lax.axis_index("c")
        sid = lax.axis_index("s")
        wid = sid * nc + cid
        r0 = sid * rps
        pltpu.sync_copy(zeros_hbm.at[pl.ds(r0, rps)], acc.at[pl.ds(r0, rps)])

        @pl.when(sid == 0)
        def _():
            pltpu.sync_copy(zeros_hbm.at[pl.ds(rps * ns, tail)],
                            acc.at[pl.ds(rps * ns, tail)])

        plsc.subcore_barrier()

        t0 = wid * blk_per_w

        def fire(t, di, rb, rsem):
            pltpu.async_copy(dst_hbm.at[pl.ds(t * EB, EB)], di, rsem)
            pltpu.async_copy(e_hbm.at[pl.ds(t * EB, EB)], rb, rsem)

        def scat(di, rb, rsem):
            pltpu.make_async_copy(dst_hbm.at[pl.ds(0, EB)], di, rsem).wait()
            pltpu.make_async_copy(e_hbm.at[pl.ds(0, EB)], rb, rsem).wait()
            pltpu.sync_copy(rb, acc.at[di], add=True)

        fire(t0, di0, rb0, r0sem)

        def body(i, _):
            fire(t0 + 2 * i + 1, di1, rb1, r1sem)
            scat(di0, rb0, r0sem)

            @pl.when(i < blk_per_w // 2 - 1)
            def _():
                fire(t0 + 2 * i + 2, di0, rb0, r0sem)

            scat(di1, rb1, r1sem)
            return 0

        lax.fori_loop(0, blk_per_w // 2, body, 0)

        @pl.when(wid < n_extra)
        def _():
            fire(nw * blk_per_w + wid, di0, rb0, r0sem)
            scat(di0, rb0, r0sem)

        plsc.subcore_barrier()
        pltpu.sync_copy(acc.at[pl.ds(r0, rps)],
                        out_hbm.at[cid, pl.ds(r0, rps)])

        @pl.when(sid == 0)
        def _():
            pltpu.sync_copy(acc.at[pl.ds(rps * ns, tail)],
                            out_hbm.at[cid, pl.ds(rps * ns, tail)])

    return scatter


def kernel(efeat, nfeat, edge_index, params):
    src = edge_index[0].astype(jnp.int32)
    dst = edge_index[1].astype(jnp.int32)

    gather = _make_gather()
    scatter = _make_scatter()
    zeros = jnp.zeros((N_NODES, D), jnp.float32)

    def prep(p):
        e, n = p['edge'], p['node']
        return dict(
            w1e=e['w1'][:D], w1s=e['w1'][D:2 * D], w1d=e['w1'][2 * D:],
            eb1=e['b1'].reshape(1, H), ew2=e['w2'],
            eb2=e['b2'].reshape(1, D), eg=e['gamma'].reshape(1, D),
            ebt=e['beta'].reshape(1, D),
            w1n=n['w1'][:D], w1a=n['w1'][D:],
            nb1=n['b1'].reshape(1, H), nw2=n['w2'],
            nb2=n['b2'].reshape(1, D), ng=n['gamma'].reshape(1, D),
            nbt=n['beta'].reshape(1, D),
        )

    ps_list = [prep(p) for p in params]
    nlayers = len(ps_list)

    ps, pd = _proj(nfeat, ps_list[0]['w1s'], ps_list[0]['w1d'])
    for l, q in enumerate(ps_list):
        gs, gd = gather(ps, pd, src, dst)
        efeat = _edge_mlp(efeat, gs, gd, q['w1e'], q['eb1'], q['ew2'],
                          q['eb2'], q['eg'], q['ebt'])
        agg2 = scatter(efeat, dst, zeros)
        nxt = ps_list[(l + 1) % nlayers]
        nfeat, ps, pd = _node_mlp(nfeat, agg2, q['w1n'], q['w1a'], q['nb1'],
                                  q['nw2'], q['nb2'], q['ng'], q['nbt'],
                                  nxt['w1s'], nxt['w1d'])
    return (efeat, nfeat)


# speedup vs baseline: 1.4999x; 1.0916x over previous
import functools

import jax
import jax.numpy as jnp
from jax import lax
from jax.experimental import pallas as pl
from jax.experimental.pallas import tpu as pltpu
from jax.experimental.pallas import tpu_sc as plsc

N_NODES = 10000
N_EDGES = 320000
D = 128
H = 128

EB = 128
NB = N_EDGES // EB


def _edge_body(ef_ref, gs_ref, gd_ref, w1_ref, b1_ref, w2_ref, b2_ref,
               gm_ref, bt_ref, out_ref):
    x = ef_ref[...]
    h = jnp.dot(x, w1_ref[...], preferred_element_type=jnp.float32)
    h = h + gs_ref[...] + gd_ref[...] + b1_ref[...]
    h = h * jax.nn.sigmoid(h)
    y = jnp.dot(h, w2_ref[...], preferred_element_type=jnp.float32) + b2_ref[...]
    mu = jnp.mean(y, axis=-1, keepdims=True)
    var = jnp.mean((y - mu) ** 2, axis=-1, keepdims=True)
    y = (y - mu) * lax.rsqrt(var + 1e-5)
    out_ref[...] = y * gm_ref[...] + bt_ref[...] + x


def _edge_mlp(ef, gs, gd, w1e, b1, w2, b2, gamma, beta, blk=4000):
    grid = (N_EDGES // blk,)
    row = lambda i: (i, 0)
    fix = lambda i: (0, 0)
    return pl.pallas_call(
        _edge_body,
        grid=grid,
        in_specs=[
            pl.BlockSpec((blk, D), row),
            pl.BlockSpec((blk, H), row),
            pl.BlockSpec((blk, H), row),
            pl.BlockSpec((D, H), fix),
            pl.BlockSpec((1, H), fix),
            pl.BlockSpec((H, D), fix),
            pl.BlockSpec((1, D), fix),
            pl.BlockSpec((1, D), fix),
            pl.BlockSpec((1, D), fix),
        ],
        out_specs=pl.BlockSpec((blk, D), row),
        out_shape=jax.ShapeDtypeStruct((N_EDGES, D), jnp.float32),
        compiler_params=pltpu.CompilerParams(
            dimension_semantics=("arbitrary",)),
    )(ef, gs, gd, w1e, b1, w2, b2, gamma, beta)


def _node_body(nf_ref, agg_ref, w1n_ref, w1a_ref, b1_ref, w2_ref, b2_ref,
               gm_ref, bt_ref, w1s_ref, w1d_ref,
               nf_out, ps_out, pd_out):
    x = nf_ref[...]
    a = agg_ref[0] + agg_ref[1]
    h = (jnp.dot(x, w1n_ref[...], preferred_element_type=jnp.float32)
         + jnp.dot(a, w1a_ref[...], preferred_element_type=jnp.float32)
         + b1_ref[...])
    h = h * jax.nn.sigmoid(h)
    y = jnp.dot(h, w2_ref[...], preferred_element_type=jnp.float32) + b2_ref[...]
    mu = jnp.mean(y, axis=-1, keepdims=True)
    var = jnp.mean((y - mu) ** 2, axis=-1, keepdims=True)
    y = (y - mu) * lax.rsqrt(var + 1e-5)
    y = y * gm_ref[...] + bt_ref[...] + x
    nf_out[...] = y
    ps_out[...] = jnp.dot(y, w1s_ref[...], preferred_element_type=jnp.float32)
    pd_out[...] = jnp.dot(y, w1d_ref[...], preferred_element_type=jnp.float32)


def _node_mlp(nf, agg2, w1n, w1a, b1, w2, b2, gamma, beta, w1s_nxt, w1d_nxt):
    out_shape = [
        jax.ShapeDtypeStruct((N_NODES, D), jnp.float32),
        jax.ShapeDtypeStruct((N_NODES, H), jnp.float32),
        jax.ShapeDtypeStruct((N_NODES, H), jnp.float32),
    ]
    return pl.pallas_call(_node_body, out_shape=out_shape)(
        nf, agg2, w1n, w1a, b1, w2, b2, gamma, beta, w1s_nxt, w1d_nxt)


def _proj_body(nf_ref, w1s_ref, w1d_ref, ps_out, pd_out):
    x = nf_ref[...]
    ps_out[...] = jnp.dot(x, w1s_ref[...], preferred_element_type=jnp.float32)
    pd_out[...] = jnp.dot(x, w1d_ref[...], preferred_element_type=jnp.float32)


def _proj(nf, w1s, w1d):
    out_shape = [
        jax.ShapeDtypeStruct((N_NODES, H), jnp.float32),
        jax.ShapeDtypeStruct((N_NODES, H), jnp.float32),
    ]
    return pl.pallas_call(_proj_body, out_shape=out_shape)(nf, w1s, w1d)


def _make_gather():
    info = plsc.get_sparse_core_info()
    ns = info.num_subcores
    mesh = plsc.VectorSubcoreMesh(core_axis_name="c", subcore_axis_name="s")
    GB = 400
    E_PER = N_EDGES // ns
    NIT = E_PER // GB
    SUB = ((0, 128), (128, 128), (256, 128), (384, 16))

    @functools.partial(
        pl.kernel,
        mesh=mesh,
        out_type=(
            jax.ShapeDtypeStruct((N_EDGES, H), jnp.float32),
            jax.ShapeDtypeStruct((N_EDGES, H), jnp.float32),
        ),
        scratch_types=[
            pltpu.VMEM((E_PER,), jnp.int32),
            pltpu.VMEM((GB, H), jnp.float32),
            pltpu.VMEM((GB, H), jnp.float32),
            pltpu.SemaphoreType.DMA,
            pltpu.SemaphoreType.DMA,
            pltpu.SemaphoreType.DMA,
            pltpu.SemaphoreType.DMA,
        ],
    )
    def gather(ps_hbm, pd_hbm, src_hbm, dst_hbm, gs_hbm, gd_hbm,
               ibig, buf0, buf1, g0, g1, w0, w1):
        cid = lax.axis_index("c")
        sid = lax.axis_index("s")
        base_e = sid * E_PER

        def run(idx_hbm, tab_hbm, out_hbm):
            pltpu.sync_copy(idx_hbm.at[pl.ds(base_e, E_PER)], ibig)

            def fire(it, buf, gsem):
                off = it * GB
                for (o, n) in SUB:
                    pltpu.async_copy(tab_hbm.at[ibig.at[pl.ds(off + o, n)]],
                                     buf.at[pl.ds(o, n)], gsem)

            def wait_g(buf, gsem):
                for (o, n) in SUB:
                    pltpu.make_async_copy(tab_hbm.at[pl.ds(0, n)],
                                          buf.at[pl.ds(o, n)], gsem).wait()

            def write(it, buf, wsem):
                pltpu.async_copy(
                    buf, out_hbm.at[pl.ds(base_e + it * GB, GB)], wsem)

            def wait_w(buf, wsem):
                pltpu.make_async_copy(out_hbm.at[pl.ds(base_e, GB)],
                                      buf, wsem).wait()

            fire(0, buf0, g0)

            def body(i, _):
                @pl.when(i > 0)
                def _():
                    wait_w(buf1, w1)

                fire(2 * i + 1, buf1, g1)
                wait_g(buf0, g0)
                write(2 * i, buf0, w0)

                @pl.when(i < NIT // 2 - 1)
                def _():
                    wait_w(buf0, w0)
                    fire(2 * i + 2, buf0, g0)

                wait_g(buf1, g1)
                write(2 * i + 1, buf1, w1)
                return 0

            lax.fori_loop(0, NIT // 2, body, 0)
            wait_w(buf0, w0)
            wait_w(buf1, w1)

        @pl.when(cid == 0)
        def _():
            run(src_hbm, ps_hbm, gs_hbm)

        @pl.when(cid == 1)
        def _():
            run(dst_hbm, pd_hbm, gd_hbm)

    return gather


def _make_scatter():
    info = plsc.get_sparse_core_info()
    nc, ns = info.num_cores, info.num_subcores
    nw = nc * ns
    rps = 624
    tail = N_NODES - rps * ns
    mesh = plsc.VectorSubcoreMesh(core_axis_name="c", subcore_axis_name="s")

    blk_per_w = NB // nw
    n_extra = NB - blk_per_w * nw

    @functools.partial(
        pl.kernel,
        mesh=mesh,
        out_type=jax.ShapeDtypeStruct((2, N_NODES, D), jnp.float32),
        scratch_types=[
            pltpu.VMEM((EB,), jnp.int32),
            pltpu.VMEM((EB,), jnp.int32),
            pltpu.VMEM((EB, D), jnp.float32),
            pltpu.VMEM((EB, D), jnp.float32),
            pltpu.VMEM_SHARED((N_NODES, D), jnp.float32),
            pltpu.SemaphoreType.DMA,
            pltpu.SemaphoreType.DMA,
        ],
    )
    def scatter(e_hbm, dst_hbm, zeros_hbm, out_hbm,
                di0, di1, rb0, rb1, acc, r0sem, r1sem):
        cid = lax.axis_index("c")
        sid = lax.axis_index("s")
        wid = sid * nc + cid
        r0 = sid * rps
        pltpu.sync_copy(zeros_hbm.at[pl.ds(r0, rps)], acc.at[pl.ds(r0, rps)])

        @pl.when(sid == 0)
        def _():
            pltpu.sync_copy(zeros_hbm.at[pl.ds(rps * ns, tail)],
                            acc.at[pl.ds(rps * ns, tail)])

        plsc.subcore_barrier()

        t0 = wid * blk_per_w

        def fire(t, di, rb, rsem):
            pltpu.async_copy(dst_hbm.at[pl.ds(t * EB, EB)], di, rsem)
            pltpu.async_copy(e_hbm.at[pl.ds(t * EB, EB)], rb, rsem)

        def scat(di, rb, rsem):
            pltpu.make_async_copy(dst_hbm.at[pl.ds(0, EB)], di, rsem).wait()
            pltpu.make_async_copy(e_hbm.at[pl.ds(0, EB)], rb, rsem).wait()
            pltpu.sync_copy(rb, acc.at[di], add=True)

        fire(t0, di0, rb0, r0sem)

        def body(i, _):
            fire(t0 + 2 * i + 1, di1, rb1, r1sem)
            scat(di0, rb0, r0sem)

            @pl.when(i < blk_per_w // 2 - 1)
            def _():
                fire(t0 + 2 * i + 2, di0, rb0, r0sem)

            scat(di1, rb1, r1sem)
            return 0

        lax.fori_loop(0, blk_per_w // 2, body, 0)

        @pl.when(wid < n_extra)
        def _():
            fire(nw * blk_per_w + wid, di0, rb0, r0sem)
            scat(di0, rb0, r0sem)

        plsc.subcore_barrier()
        pltpu.sync_copy(acc.at[pl.ds(r0, rps)],
                        out_hbm.at[cid, pl.ds(r0, rps)])

        @pl.when(sid == 0)
        def _():
            pltpu.sync_copy(acc.at[pl.ds(rps * ns, tail)],
                            out_hbm.at[cid, pl.ds(rps * ns, tail)])

    return scatter


def kernel(efeat, nfeat, edge_index, params):
    src = edge_index[0].astype(jnp.int32)
    dst = edge_index[1].astype(jnp.int32)

    gather = _make_gather()
    scatter = _make_scatter()
    zeros = jnp.zeros((N_NODES, D), jnp.float32)

    def prep(p):
        e, n = p['edge'], p['node']
        return dict(
            w1e=e['w1'][:D], w1s=e['w1'][D:2 * D], w1d=e['w1'][2 * D:],
            eb1=e['b1'].reshape(1, H), ew2=e['w2'],
            eb2=e['b2'].reshape(1, D), eg=e['gamma'].reshape(1, D),
            ebt=e['beta'].reshape(1, D),
            w1n=n['w1'][:D], w1a=n['w1'][D:],
            nb1=n['b1'].reshape(1, H), nw2=n['w2'],
            nb2=n['b2'].reshape(1, D), ng=n['gamma'].reshape(1, D),
            nbt=n['beta'].reshape(1, D),
        )

    ps_list = [prep(p) for p in params]
    nlayers = len(ps_list)

    ps, pd = _proj(nfeat, ps_list[0]['w1s'], ps_list[0]['w1d'])
    for l, q in enumerate(ps_list):
        gs, gd = gather(ps, pd, src, dst)
        efeat = _edge_mlp(efeat, gs, gd, q['w1e'], q['eb1'], q['ew2'],
                          q['eb2'], q['eg'], q['ebt'])
        agg2 = scatter(efeat, dst, zeros)
        nxt = ps_list[(l + 1) % nlayers]
        nfeat, ps, pd = _node_mlp(nfeat, agg2, q['w1n'], q['w1a'], q['nb1'],
                                  q['nw2'], q['nb2'], q['ng'], q['nbt'],
                                  nxt['w1s'], nxt['w1d'])
    return (efeat, nfeat)


# speedup vs baseline: 1.5209x; 1.0140x over previous
import functools

import jax
import jax.numpy as jnp
from jax import lax
from jax.experimental import pallas as pl
from jax.experimental.pallas import tpu as pltpu
from jax.experimental.pallas import tpu_sc as plsc

N_NODES = 10000
N_EDGES = 320000
D = 128
H = 128

EB = 128
NB = N_EDGES // EB


def _edge_body(ef_ref, gs_ref, gd_ref, w1_ref, b1_ref, w2_ref, b2_ref,
               gm_ref, bt_ref, out_ref):
    x = ef_ref[...]
    h = jnp.dot(x, w1_ref[...], preferred_element_type=jnp.float32)
    h = h + gs_ref[...] + gd_ref[...] + b1_ref[...]
    h = h * jax.nn.sigmoid(h)
    y = jnp.dot(h, w2_ref[...], preferred_element_type=jnp.float32) + b2_ref[...]
    mu = jnp.mean(y, axis=-1, keepdims=True)
    var = jnp.mean((y - mu) ** 2, axis=-1, keepdims=True)
    y = (y - mu) * lax.rsqrt(var + 1e-5)
    out_ref[...] = y * gm_ref[...] + bt_ref[...] + x


def _edge_mlp(ef, gs, gd, w1e, b1, w2, b2, gamma, beta, blk=8000):
    grid = (N_EDGES // blk,)
    row = lambda i: (i, 0)
    fix = lambda i: (0, 0)
    return pl.pallas_call(
        _edge_body,
        grid=grid,
        in_specs=[
            pl.BlockSpec((blk, D), row),
            pl.BlockSpec((blk, H), row),
            pl.BlockSpec((blk, H), row),
            pl.BlockSpec((D, H), fix),
            pl.BlockSpec((1, H), fix),
            pl.BlockSpec((H, D), fix),
            pl.BlockSpec((1, D), fix),
            pl.BlockSpec((1, D), fix),
            pl.BlockSpec((1, D), fix),
        ],
        out_specs=pl.BlockSpec((blk, D), row),
        out_shape=jax.ShapeDtypeStruct((N_EDGES, D), jnp.float32),
        compiler_params=pltpu.CompilerParams(
            dimension_semantics=("arbitrary",)),
    )(ef, gs, gd, w1e, b1, w2, b2, gamma, beta)


def _node_body(nf_ref, agg_ref, w1n_ref, w1a_ref, b1_ref, w2_ref, b2_ref,
               gm_ref, bt_ref, w1s_ref, w1d_ref,
               nf_out, ps_out, pd_out):
    x = nf_ref[...]
    a = agg_ref[0] + agg_ref[1]
    h = (jnp.dot(x, w1n_ref[...], preferred_element_type=jnp.float32)
         + jnp.dot(a, w1a_ref[...], preferred_element_type=jnp.float32)
         + b1_ref[...])
    h = h * jax.nn.sigmoid(h)
    y = jnp.dot(h, w2_ref[...], preferred_element_type=jnp.float32) + b2_ref[...]
    mu = jnp.mean(y, axis=-1, keepdims=True)
    var = jnp.mean((y - mu) ** 2, axis=-1, keepdims=True)
    y = (y - mu) * lax.rsqrt(var + 1e-5)
    y = y * gm_ref[...] + bt_ref[...] + x
    nf_out[...] = y
    ps_out[...] = jnp.dot(y, w1s_ref[...], preferred_element_type=jnp.float32)
    pd_out[...] = jnp.dot(y, w1d_ref[...], preferred_element_type=jnp.float32)


def _node_mlp(nf, agg2, w1n, w1a, b1, w2, b2, gamma, beta, w1s_nxt, w1d_nxt):
    out_shape = [
        jax.ShapeDtypeStruct((N_NODES, D), jnp.float32),
        jax.ShapeDtypeStruct((N_NODES, H), jnp.float32),
        jax.ShapeDtypeStruct((N_NODES, H), jnp.float32),
    ]
    return pl.pallas_call(_node_body, out_shape=out_shape)(
        nf, agg2, w1n, w1a, b1, w2, b2, gamma, beta, w1s_nxt, w1d_nxt)


def _proj_body(nf_ref, w1s_ref, w1d_ref, ps_out, pd_out):
    x = nf_ref[...]
    ps_out[...] = jnp.dot(x, w1s_ref[...], preferred_element_type=jnp.float32)
    pd_out[...] = jnp.dot(x, w1d_ref[...], preferred_element_type=jnp.float32)


def _proj(nf, w1s, w1d):
    out_shape = [
        jax.ShapeDtypeStruct((N_NODES, H), jnp.float32),
        jax.ShapeDtypeStruct((N_NODES, H), jnp.float32),
    ]
    return pl.pallas_call(_proj_body, out_shape=out_shape)(nf, w1s, w1d)


def _make_gather():
    info = plsc.get_sparse_core_info()
    ns = info.num_subcores
    mesh = plsc.VectorSubcoreMesh(core_axis_name="c", subcore_axis_name="s")
    GB = 400
    E_PER = N_EDGES // ns
    NIT = E_PER // GB
    SUB = ((0, 128), (128, 128), (256, 128), (384, 16))

    @functools.partial(
        pl.kernel,
        mesh=mesh,
        out_type=(
            jax.ShapeDtypeStruct((N_EDGES, H), jnp.float32),
            jax.ShapeDtypeStruct((N_EDGES, H), jnp.float32),
        ),
        scratch_types=[
            pltpu.VMEM((E_PER,), jnp.int32),
            pltpu.VMEM((GB, H), jnp.float32),
            pltpu.VMEM((GB, H), jnp.float32),
            pltpu.SemaphoreType.DMA,
            pltpu.SemaphoreType.DMA,
            pltpu.SemaphoreType.DMA,
            pltpu.SemaphoreType.DMA,
        ],
    )
    def gather(ps_hbm, pd_hbm, src_hbm, dst_hbm, gs_hbm, gd_hbm,
               ibig, buf0, buf1, g0, g1, w0, w1):
        cid = lax.axis_index("c")
        sid = lax.axis_index("s")
        base_e = sid * E_PER

        def run(idx_hbm, tab_hbm, out_hbm):
            pltpu.sync_copy(idx_hbm.at[pl.ds(base_e, E_PER)], ibig)

            def fire(it, buf, gsem):
                off = it * GB
                for (o, n) in SUB:
                    pltpu.async_copy(tab_hbm.at[ibig.at[pl.ds(off + o, n)]],
                                     buf.at[pl.ds(o, n)], gsem)

            def wait_g(buf, gsem):
                for (o, n) in SUB:
                    pltpu.make_async_copy(tab_hbm.at[pl.ds(0, n)],
                                          buf.at[pl.ds(o, n)], gsem).wait()

            def write(it, buf, wsem):
                pltpu.async_copy(
                    buf, out_hbm.at[pl.ds(base_e + it * GB, GB)], wsem)

            def wait_w(buf, wsem):
                pltpu.make_async_copy(out_hbm.at[pl.ds(base_e, GB)],
                                      buf, wsem).wait()

            fire(0, buf0, g0)

            def body(i, _):
                @pl.when(i > 0)
                def _():
                    wait_w(buf1, w1)

                fire(2 * i + 1, buf1, g1)
                wait_g(buf0, g0)
                write(2 * i, buf0, w0)

                @pl.when(i < NIT // 2 - 1)
                def _():
                    wait_w(buf0, w0)
                    fire(2 * i + 2, buf0, g0)

                wait_g(buf1, g1)
                write(2 * i + 1, buf1, w1)
                return 0

            lax.fori_loop(0, NIT // 2, body, 0)
            wait_w(buf0, w0)
            wait_w(buf1, w1)

        @pl.when(cid == 0)
        def _():
            run(src_hbm, ps_hbm, gs_hbm)

        @pl.when(cid == 1)
        def _():
            run(dst_hbm, pd_hbm, gd_hbm)

    return gather


def _make_scatter():
    info = plsc.get_sparse_core_info()
    nc, ns = info.num_cores, info.num_subcores
    nw = nc * ns
    rps = 624
    tail = N_NODES - rps * ns
    mesh = plsc.VectorSubcoreMesh(core_axis_name="c", subcore_axis_name="s")

    blk_per_w = NB // nw
    n_extra = NB - blk_per_w * nw

    @functools.partial(
        pl.kernel,
        mesh=mesh,
        out_type=jax.ShapeDtypeStruct((2, N_NODES, D), jnp.float32),
        scratch_types=[
            pltpu.VMEM((EB,), jnp.int32),
            pltpu.VMEM((EB,), jnp.int32),
            pltpu.VMEM((EB, D), jnp.float32),
            pltpu.VMEM((EB, D), jnp.float32),
            pltpu.VMEM_SHARED((N_NODES, D), jnp.float32),
            pltpu.SemaphoreType.DMA,
            pltpu.SemaphoreType.DMA,
        ],
    )
    def scatter(e_hbm, dst_hbm, zeros_hbm, out_hbm,
                di0, di1, rb0, rb1, acc, r0sem, r1sem):
        cid = lax.axis_index("c")
        sid = lax.axis_index("s")
        wid = sid * nc + cid
        r0 = sid * rps
        pltpu.sync_copy(zeros_hbm.at[pl.ds(r0, rps)], acc.at[pl.ds(r0, rps)])

        @pl.when(sid == 0)
        def _():
            pltpu.sync_copy(zeros_hbm.at[pl.ds(rps * ns, tail)],
                            acc.at[pl.ds(rps * ns, tail)])

        plsc.subcore_barrier()

        t0 = wid * blk_per_w

        def fire(t, di, rb, rsem):
            pltpu.async_copy(dst_hbm.at[pl.ds(t * EB, EB)], di, rsem)
            pltpu.async_copy(e_hbm.at[pl.ds(t * EB, EB)], rb, rsem)

        def scat(di, rb, rsem):
            pltpu.make_async_copy(dst_hbm.at[pl.ds(0, EB)], di, rsem).wait()
            pltpu.make_async_copy(e_hbm.at[pl.ds(0, EB)], rb, rsem).wait()
            pltpu.sync_copy(rb, acc.at[di], add=True)

        fire(t0, di0, rb0, r0sem)

        def body(i, _):
            fire(t0 + 2 * i + 1, di1, rb1, r1sem)
            scat(di0, rb0, r0sem)

            @pl.when(i < blk_per_w // 2 - 1)
            def _():
                fire(t0 + 2 * i + 2, di0, rb0, r0sem)

            scat(di1, rb1, r1sem)
            return 0

        lax.fori_loop(0, blk_per_w // 2, body, 0)

        @pl.when(wid < n_extra)
        def _():
            fire(nw * blk_per_w + wid, di0, rb0, r0sem)
            scat(di0, rb0, r0sem)

        plsc.subcore_barrier()
        pltpu.sync_copy(acc.at[pl.ds(r0, rps)],
                        out_hbm.at[cid, pl.ds(r0, rps)])

        @pl.when(sid == 0)
        def _():
            pltpu.sync_copy(acc.at[pl.ds(rps * ns, tail)],
                            out_hbm.at[cid, pl.ds(rps * ns, tail)])

    return scatter


def kernel(efeat, nfeat, edge_index, params):
    src = edge_index[0].astype(jnp.int32)
    dst = edge_index[1].astype(jnp.int32)

    gather = _make_gather()
    scatter = _make_scatter()
    zeros = jnp.zeros((N_NODES, D), jnp.float32)

    def prep(p):
        e, n = p['edge'], p['node']
        return dict(
            w1e=e['w1'][:D], w1s=e['w1'][D:2 * D], w1d=e['w1'][2 * D:],
            eb1=e['b1'].reshape(1, H), ew2=e['w2'],
            eb2=e['b2'].reshape(1, D), eg=e['gamma'].reshape(1, D),
            ebt=e['beta'].reshape(1, D),
            w1n=n['w1'][:D], w1a=n['w1'][D:],
            nb1=n['b1'].reshape(1, H), nw2=n['w2'],
            nb2=n['b2'].reshape(1, D), ng=n['gamma'].reshape(1, D),
            nbt=n['beta'].reshape(1, D),
        )

    ps_list = [prep(p) for p in params]
    nlayers = len(ps_list)

    ps, pd = _proj(nfeat, ps_list[0]['w1s'], ps_list[0]['w1d'])
    for l, q in enumerate(ps_list):
        gs, gd = gather(ps, pd, src, dst)
        efeat = _edge_mlp(efeat, gs, gd, q['w1e'], q['eb1'], q['ew2'],
                          q['eb2'], q['eg'], q['ebt'])
        agg2 = scatter(efeat, dst, zeros)
        nxt = ps_list[(l + 1) % nlayers]
        nfeat, ps, pd = _node_mlp(nfeat, agg2, q['w1n'], q['w1a'], q['nb1'],
                                  q['nw2'], q['nb2'], q['ng'], q['nbt'],
                                  nxt['w1s'], nxt['w1d'])
    return (efeat, nfeat)
